# Initial kernel scaffold; baseline (speedup 1.0000x reference)
import jax
import jax.numpy as jnp
from jax.experimental import pallas as pl


def kernel(charges, cell, positions, neighbor_indices, neighbor_distances):
    raise NotImplementedError("write your pallas kernel here")


# speedup vs baseline: 36.9702x; 36.9702x over previous
import functools

import jax
import jax.numpy as jnp
from jax import lax
from jax.experimental import pallas as pl
from jax.experimental.pallas import tpu as pltpu
from jax.experimental.pallas import tpu_sc as plsc

N_CORES = 2
N_SUBCORES = 16
N_TILES = N_CORES * N_SUBCORES
LANES = 16
CHUNK = 4000
CH4 = CHUNK * 4


def _sc_accumulate(charges_flat, idx_i, idx_j, dists):
    nflat = charges_flat.shape[0]
    npairs = dists.shape[0]
    ppt = npairs // N_TILES
    nchunks = ppt // CHUNK
    tile_elems = nflat // N_SUBCORES

    mesh = plsc.VectorSubcoreMesh(core_axis_name="c", subcore_axis_name="s")

    @functools.partial(
        pl.kernel,
        mesh=mesh,
        compiler_params=pltpu.CompilerParams(
            needs_layout_passes=False, use_tc_tiling_on_sc=False),
        out_type=jax.ShapeDtypeStruct((N_CORES * nflat,), jnp.float32),
        scratch_types=[
            pltpu.VMEM_SHARED((nflat,), jnp.float32),
            pltpu.VMEM_SHARED((nflat,), jnp.float32),
            pltpu.VMEM((CHUNK,), jnp.int32),
            pltpu.VMEM((CHUNK,), jnp.int32),
            pltpu.VMEM((CHUNK,), jnp.float32),
            pltpu.VMEM((CH4,), jnp.int32),
            pltpu.VMEM((CH4,), jnp.int32),
            pltpu.VMEM((CH4,), jnp.float32),
            pltpu.VMEM((CH4,), jnp.float32),
        ],
    )
    def run(ch_hbm, ii_hbm, jj_hbm, dd_hbm, out_hbm,
            ch_sp, acc_sp, ii_v, jj_v, dd_v, ii4_v, jj4_v, val_i, val_j):
        c = lax.axis_index("c")
        s = lax.axis_index("s")
        wid = s * N_CORES + c
        lane = lax.iota(jnp.int32, LANES)
        l4 = lane >> 2
        lm = lane & 3
        zeros = jnp.zeros((LANES,), jnp.float32)

        def zbody(k, carry):
            val_j[pl.ds(k * LANES, LANES)] = zeros
            return carry
        lax.fori_loop(0, CH4 // LANES, zbody, 0)

        e0 = s * tile_elems

        def over_slices(total, fn):
            off = 0
            while off < total:
                n = min(CH4, total - off)
                fn(off, n)
                off += n

        def init(off, n):
            pltpu.sync_copy(val_j.at[pl.ds(0, n)],
                            acc_sp.at[pl.ds(e0 + off, n)])
            pltpu.sync_copy(ch_hbm.at[pl.ds(e0 + off, n)],
                            val_i.at[pl.ds(0, n)])
            pltpu.sync_copy(val_i.at[pl.ds(0, n)],
                            ch_sp.at[pl.ds(e0 + off, n)])
        over_slices(tile_elems, init)

        plsc.subcore_barrier()

        def chunk_body(t, carry):
            base = wid * ppt + t * CHUNK
            pltpu.sync_copy(ii_hbm.at[pl.ds(base, CHUNK)], ii_v)
            pltpu.sync_copy(jj_hbm.at[pl.ds(base, CHUNK)], jj_v)
            pltpu.sync_copy(dd_hbm.at[pl.ds(base, CHUNK)], dd_v)

            def abody(k, cr):
                pidx = 4 * k + l4
                sl = pl.ds(k * LANES, LANES)
                i16 = plsc.load_gather(ii_v, [pidx])
                j16 = plsc.load_gather(jj_v, [pidx])
                ii4_v[sl] = 4 * i16 + lm
                jj4_v[sl] = 4 * j16 + lm
                return cr
            lax.fori_loop(0, CH4 // LANES, abody, 0)

            def rbody(k, cr):
                sl = pl.ds(k * LANES, LANES)
                dd_v[sl] = 0.5 / dd_v[sl]
                return cr
            lax.fori_loop(0, CHUNK // LANES, rbody, 0)

            pltpu.sync_copy(ch_sp.at[jj4_v], val_j)
            pltpu.sync_copy(ch_sp.at[ii4_v], val_i)

            def mbody(k, cr):
                pidx = 4 * k + l4
                p = plsc.load_gather(dd_v, [pidx])
                sl = pl.ds(k * LANES, LANES)
                val_j[sl] = val_j[sl] * p
                val_i[sl] = val_i[sl] * p
                return cr
            lax.fori_loop(0, CH4 // LANES, mbody, 0)

            pltpu.sync_copy(val_j, acc_sp.at[ii4_v], add=True)
            pltpu.sync_copy(val_i, acc_sp.at[jj4_v], add=True)
            return carry
        lax.fori_loop(0, nchunks, chunk_body, 0)

        plsc.subcore_barrier()

        def write(off, n):
            pltpu.sync_copy(acc_sp.at[pl.ds(e0 + off, n)],
                            val_i.at[pl.ds(0, n)])
            pltpu.sync_copy(val_i.at[pl.ds(0, n)],
                            out_hbm.at[pl.ds(c * nflat + e0 + off, n)])
        over_slices(tile_elems, write)

    return run(charges_flat, idx_i, idx_j, dists)


def _tc_add(a, b):
    def body(a_ref, b_ref, o_ref):
        o_ref[...] = a_ref[...] + b_ref[...]
    return pl.pallas_call(
        body, out_shape=jax.ShapeDtypeStruct(a.shape, a.dtype))(a, b)


def kernel(charges, cell, positions, neighbor_indices, neighbor_distances):
    na, ch = charges.shape
    nflat = na * ch
    idx_i = neighbor_indices[:, 0]
    idx_j = neighbor_indices[:, 1]
    parts = _sc_accumulate(charges.reshape(nflat), idx_i, idx_j,
                           neighbor_distances)
    p0 = parts[:nflat].reshape(nflat // 128, 128)
    p1 = parts[nflat:].reshape(nflat // 128, 128)
    return _tc_add(p0, p1).reshape(na, ch)


# speedup vs baseline: 52.0516x; 1.4079x over previous
import functools

import jax
import jax.numpy as jnp
from jax import lax
from jax.experimental import pallas as pl
from jax.experimental.pallas import tpu as pltpu
from jax.experimental.pallas import tpu_sc as plsc

N_CORES = 2
N_SUBCORES = 16
N_TILES = N_CORES * N_SUBCORES
LANES = 16
ROWW = 8
CHUNK = 1600


def _sc_accumulate(charges8, idx_i, idx_j, dists):
    na = charges8.shape[0]
    npairs = dists.shape[0]
    ppt = npairs // N_TILES
    nchunks = ppt // CHUNK
    trows = na // N_SUBCORES

    mesh = plsc.VectorSubcoreMesh(core_axis_name="c", subcore_axis_name="s")

    @functools.partial(
        pl.kernel,
        mesh=mesh,
        compiler_params=pltpu.CompilerParams(
            needs_layout_passes=False, use_tc_tiling_on_sc=False),
        out_type=jax.ShapeDtypeStruct((N_CORES * na, ROWW), jnp.float32),
        scratch_types=[
            pltpu.VMEM_SHARED((na, ROWW), jnp.float32),
            pltpu.VMEM_SHARED((na, ROWW), jnp.float32),
            pltpu.VMEM((CHUNK,), jnp.int32),
            pltpu.VMEM((CHUNK,), jnp.int32),
            pltpu.VMEM((CHUNK,), jnp.float32),
            pltpu.VMEM((CHUNK, ROWW), jnp.float32),
            pltpu.VMEM((CHUNK, ROWW), jnp.float32),
        ],
    )
    def run(ch_hbm, ii_hbm, jj_hbm, dd_hbm, out_hbm,
            ch_sp, acc_sp, ii_v, jj_v, dd_v, val_i, val_j):
        c = lax.axis_index("c")
        s = lax.axis_index("s")
        wid = s * N_CORES + c
        lane = lax.iota(jnp.int32, LANES)
        l4 = lane >> 2
        lm = lane & 3
        l8 = lane >> 3
        lw = lane & 7
        zeros = jnp.zeros((LANES,), jnp.float32)

        def zbody(k, carry):
            plsc.store_scatter(val_j, [2 * k + l8, lw], zeros)
            return carry
        lax.fori_loop(0, CHUNK * ROWW // LANES, zbody, 0)

        row0 = s * trows

        def over_slices(total, fn):
            off = 0
            while off < total:
                n = min(CHUNK, total - off)
                fn(off, n)
                off += n

        def init(off, n):
            pltpu.sync_copy(val_j.at[pl.ds(0, n)],
                            acc_sp.at[pl.ds(row0 + off, n)])
            pltpu.sync_copy(ch_hbm.at[pl.ds(row0 + off, n)],
                            val_i.at[pl.ds(0, n)])
            pltpu.sync_copy(val_i.at[pl.ds(0, n)],
                            ch_sp.at[pl.ds(row0 + off, n)])
        over_slices(trows, init)

        plsc.subcore_barrier()

        def chunk_body(t, carry):
            base = wid * ppt + t * CHUNK
            pltpu.sync_copy(ii_hbm.at[pl.ds(base, CHUNK)], ii_v)
            pltpu.sync_copy(jj_hbm.at[pl.ds(base, CHUNK)], jj_v)
            pltpu.sync_copy(dd_hbm.at[pl.ds(base, CHUNK)], dd_v)

            pltpu.sync_copy(ch_sp.at[jj_v], val_j)
            pltpu.sync_copy(ch_sp.at[ii_v], val_i)

            def mbody(k, cr):
                pidx = 4 * k + l4
                p = 0.5 / plsc.load_gather(dd_v, [pidx])
                rj = plsc.load_gather(val_j, [pidx, lm])
                ri = plsc.load_gather(val_i, [pidx, lm])
                plsc.store_scatter(val_j, [pidx, lm], rj * p)
                plsc.store_scatter(val_i, [pidx, lm], ri * p)
                return cr
            lax.fori_loop(0, CHUNK * 4 // LANES, mbody, 0)

            pltpu.sync_copy(val_j, acc_sp.at[ii_v], add=True)
            pltpu.sync_copy(val_i, acc_sp.at[jj_v], add=True)
            return carry
        lax.fori_loop(0, nchunks, chunk_body, 0)

        plsc.subcore_barrier()

        def write(off, n):
            pltpu.sync_copy(acc_sp.at[pl.ds(row0 + off, n)],
                            val_i.at[pl.ds(0, n)])
            pltpu.sync_copy(val_i.at[pl.ds(0, n)],
                            out_hbm.at[pl.ds(c * na + row0 + off, n)])
        over_slices(trows, write)

    return run(charges8, idx_i, idx_j, dists)


def _tc_add(a, b):
    def body(a_ref, b_ref, o_ref):
        o_ref[...] = a_ref[...] + b_ref[...]
    return pl.pallas_call(
        body, out_shape=jax.ShapeDtypeStruct(a.shape, a.dtype))(a, b)


def kernel(charges, cell, positions, neighbor_indices, neighbor_distances):
    na, ch = charges.shape
    idx_i = neighbor_indices[:, 0]
    idx_j = neighbor_indices[:, 1]
    charges8 = jnp.pad(charges, ((0, 0), (0, ROWW - ch)))
    parts = _sc_accumulate(charges8, idx_i, idx_j, neighbor_distances)
    p0 = parts[:na].reshape(na * ROWW // 128, 128)
    p1 = parts[na:].reshape(na * ROWW // 128, 128)
    return _tc_add(p0, p1).reshape(na, ROWW)[:, :ch]


# speedup vs baseline: 59.5638x; 1.1443x over previous
import functools

import jax
import jax.numpy as jnp
from jax import lax
from jax.experimental import pallas as pl
from jax.experimental.pallas import tpu as pltpu
from jax.experimental.pallas import tpu_sc as plsc

N_CORES = 2
N_SUBCORES = 16
N_TILES = N_CORES * N_SUBCORES
LANES = 16
ROWW = 8
CHUNK = 800
NBUF = 2


def _sc_accumulate(charges8, idx_i, idx_j, dists):
    na = charges8.shape[0]
    npairs = dists.shape[0]
    ppt = npairs // N_TILES
    nchunks = ppt // CHUNK
    trows = na // N_SUBCORES

    mesh = plsc.VectorSubcoreMesh(core_axis_name="c", subcore_axis_name="s")

    @functools.partial(
        pl.kernel,
        mesh=mesh,
        compiler_params=pltpu.CompilerParams(
            needs_layout_passes=False, use_tc_tiling_on_sc=False),
        out_type=jax.ShapeDtypeStruct((N_CORES * na, ROWW), jnp.float32),
        scratch_types=[
            pltpu.VMEM_SHARED((na, ROWW), jnp.float32),
            pltpu.VMEM_SHARED((na, ROWW), jnp.float32),
            pltpu.VMEM((NBUF, CHUNK), jnp.int32),
            pltpu.VMEM((NBUF, CHUNK), jnp.int32),
            pltpu.VMEM((NBUF, CHUNK), jnp.float32),
            pltpu.VMEM((NBUF, CHUNK, ROWW), jnp.float32),
            pltpu.VMEM((NBUF, CHUNK, ROWW), jnp.float32),
            pltpu.SemaphoreType.DMA,
            pltpu.SemaphoreType.DMA,
            pltpu.SemaphoreType.DMA,
            pltpu.SemaphoreType.DMA,
        ],
    )
    def run(ch_hbm, ii_hbm, jj_hbm, dd_hbm, out_hbm,
            ch_sp, acc_sp, ii_v, jj_v, dd_v, val_i, val_j,
            semL, semG, semS0, semS1):
        c = lax.axis_index("c")
        s = lax.axis_index("s")
        wid = s * N_CORES + c
        lane = lax.iota(jnp.int32, LANES)
        l4 = lane >> 2
        lm = lane & 3
        l8 = lane >> 3
        lw = lane & 7
        zeros = jnp.zeros((LANES,), jnp.float32)
        semS = (semS0, semS1)

        zbuf = val_j.at[0]
        sbuf = val_i.at[0]

        def zbody(k, carry):
            plsc.store_scatter(zbuf, [2 * k + l8, lw], zeros)
            return carry
        lax.fori_loop(0, CHUNK * ROWW // LANES, zbody, 0)

        row0 = s * trows

        def over_slices(total, fn):
            off = 0
            while off < total:
                n = min(CHUNK, total - off)
                fn(off, n)
                off += n

        def init(off, n):
            pltpu.sync_copy(zbuf.at[pl.ds(0, n)],
                            acc_sp.at[pl.ds(row0 + off, n)])
            pltpu.sync_copy(ch_hbm.at[pl.ds(row0 + off, n)],
                            sbuf.at[pl.ds(0, n)])
            pltpu.sync_copy(sbuf.at[pl.ds(0, n)],
                            ch_sp.at[pl.ds(row0 + off, n)])
        over_slices(trows, init)

        plsc.subcore_barrier()

        def scat_desc(b):
            return (pltpu.make_async_copy(val_j.at[b], acc_sp.at[ii_v.at[b]],
                                          semS[b]),
                    pltpu.make_async_copy(val_i.at[b], acc_sp.at[jj_v.at[b]],
                                          semS[b]))

        def process(t, b):
            base = wid * ppt + t * CHUNK
            @pl.when(t >= NBUF)
            def _():
                d1, d2 = scat_desc(b)
                d1.wait()
                d2.wait()
            ld1 = pltpu.async_copy(ii_hbm.at[pl.ds(base, CHUNK)],
                                   ii_v.at[b], semL)
            ld2 = pltpu.async_copy(jj_hbm.at[pl.ds(base, CHUNK)],
                                   jj_v.at[b], semL)
            ld3 = pltpu.async_copy(dd_hbm.at[pl.ds(base, CHUNK)],
                                   dd_v.at[b], semL)
            ld1.wait()
            ld2.wait()
            ld3.wait()
            g1 = pltpu.async_copy(ch_sp.at[jj_v.at[b]], val_j.at[b], semG)
            g2 = pltpu.async_copy(ch_sp.at[ii_v.at[b]], val_i.at[b], semG)
            g1.wait()
            g2.wait()

            vj = val_j.at[b]
            vi = val_i.at[b]
            dv = dd_v.at[b]

            def mbody(k, cr):
                pidx = 4 * k + l4
                p = 0.5 / plsc.load_gather(dv, [pidx])
                rj = plsc.load_gather(vj, [pidx, lm])
                ri = plsc.load_gather(vi, [pidx, lm])
                plsc.store_scatter(vj, [pidx, lm], rj * p)
                plsc.store_scatter(vi, [pidx, lm], ri * p)
                return cr
            lax.fori_loop(0, CHUNK * 4 // LANES, mbody, 0)

            d1, d2 = scat_desc(b)
            d1.start(add=True)
            d2.start(add=True)

        def outer(g, carry):
            for b in range(NBUF):
                process(g * NBUF + b, b)
            return carry
        lax.fori_loop(0, nchunks // NBUF, outer, 0)

        for b in range(NBUF):
            d1, d2 = scat_desc(b)
            d1.wait()
            d2.wait()

        plsc.subcore_barrier()

        def write(off, n):
            pltpu.sync_copy(acc_sp.at[pl.ds(row0 + off, n)],
                            sbuf.at[pl.ds(0, n)])
            pltpu.sync_copy(sbuf.at[pl.ds(0, n)],
                            out_hbm.at[pl.ds(c * na + row0 + off, n)])
        over_slices(trows, write)

    return run(charges8, idx_i, idx_j, dists)


def _tc_add(a, b):
    def body(a_ref, b_ref, o_ref):
        o_ref[...] = a_ref[...] + b_ref[...]
    return pl.pallas_call(
        body, out_shape=jax.ShapeDtypeStruct(a.shape, a.dtype))(a, b)


def kernel(charges, cell, positions, neighbor_indices, neighbor_distances):
    na, ch = charges.shape
    idx_i = neighbor_indices[:, 0]
    idx_j = neighbor_indices[:, 1]
    charges8 = jnp.pad(charges, ((0, 0), (0, ROWW - ch)))
    parts = _sc_accumulate(charges8, idx_i, idx_j, neighbor_distances)
    p0 = parts[:na].reshape(na * ROWW // 128, 128)
    p1 = parts[na:].reshape(na * ROWW // 128, 128)
    return _tc_add(p0, p1).reshape(na, ROWW)[:, :ch]
